# baseline (speedup 1.0000x reference)
import functools

import jax
import jax.numpy as jnp
from jax import lax
from jax.experimental import pallas as pl
from jax.experimental.pallas import tpu as pltpu
from jax.experimental.pallas import tpu_sc as plsc

NUM_TOKENS = 1000000
D = 64
B = 4096
L = 200

NC = 2
NS = 16
NW = NC * NS
B_PER_W = B // NW
CB = 4
N_CHUNKS = B_PER_W // CB
IDX_PER_CHUNK = CB * L
LP = 256
GROUPS = (0, 128)

_mesh = plsc.VectorSubcoreMesh(core_axis_name="c", subcore_axis_name="s")


@functools.partial(
    pl.kernel,
    mesh=_mesh,
    out_type=jax.ShapeDtypeStruct((B, D), jnp.float32),
    compiler_params=pltpu.CompilerParams(use_tc_tiling_on_sc=False),
    scratch_types=[
        pltpu.VMEM((CB, LP), jnp.int32),
        pltpu.VMEM((CB, LP), jnp.int32),
        pltpu.VMEM((IDX_PER_CHUNK, D), jnp.float32),
        pltpu.VMEM((IDX_PER_CHUNK, D), jnp.float32),
        pltpu.VMEM((B_PER_W, D), jnp.float32),
        pltpu.SemaphoreType.DMA,
        pltpu.SemaphoreType.DMA,
    ],
)
def _emb_kernel(idx_hbm, table_hbm, out_hbm, idx0_v, idx1_v, rows0_v, rows1_v,
                out_v, sem0, sem1):
    wid = lax.axis_index("s") * NC + lax.axis_index("c")
    row0 = wid * B_PER_W

    def gathers(idx_v, rows_v, sem):
        for s in range(CB):
            for go in GROUPS:
                gs = min(L, 128 if go == 0 else L - go)
                yield (
                    table_hbm.at[idx_v.at[s, pl.ds(go, gs)]],
                    rows_v.at[pl.ds(s * L + go, gs)],
                    sem,
                )

    def stage(g, idx_v, rows_v, sem):
        pltpu.sync_copy(idx_hbm.at[pl.ds(row0 + g * CB, CB), :], idx_v)
        for args in gathers(idx_v, rows_v, sem):
            pltpu.async_copy(*args)

    def drain(idx_v, rows_v, sem):
        for args in gathers(idx_v, rows_v, sem):
            pltpu.make_async_copy(*args).wait()

    def reduce_chunk(g, rows_v):
        for s in range(CB):
            def red(t, accs, s=s):
                base = s * L + t * 8
                a0, a1, a2, a3 = accs
                for u in range(8):
                    r = base + u
                    a0 = a0 + rows_v[r, pl.ds(0, 16)]
                    a1 = a1 + rows_v[r, pl.ds(16, 16)]
                    a2 = a2 + rows_v[r, pl.ds(32, 16)]
                    a3 = a3 + rows_v[r, pl.ds(48, 16)]
                return (a0, a1, a2, a3)
            accs = lax.fori_loop(
                0, L // 8, red,
                tuple(jnp.zeros((16,), jnp.float32) for _ in range(D // 16)),
            )
            for j in range(D // 16):
                out_v[g * CB + s, pl.ds(j * 16, 16)] = accs[j]

    stage(0, idx0_v, rows0_v, sem0)

    def pair(h, _):
        g0 = h * 2
        stage(g0 + 1, idx1_v, rows1_v, sem1)
        drain(idx0_v, rows0_v, sem0)
        reduce_chunk(g0, rows0_v)

        @pl.when(h < N_CHUNKS // 2 - 1)
        def _prefetch():
            stage(g0 + 2, idx0_v, rows0_v, sem0)

        drain(idx1_v, rows1_v, sem1)
        reduce_chunk(g0 + 1, rows1_v)
        return _

    lax.fori_loop(0, N_CHUNKS // 2, pair, None)
    pltpu.sync_copy(out_v, out_hbm.at[pl.ds(wid * B_PER_W, B_PER_W)])


def kernel(inputs, table):
    idx_pad = jnp.pad(inputs.astype(jnp.int32), ((0, 0), (0, LP - L)))
    t2 = lax.optimization_barrier(table.reshape(NUM_TOKENS // 2, 2 * D))
    return _emb_kernel(idx_pad, t2.reshape(NUM_TOKENS, D))


# speedup vs baseline: 2.0588x; 2.0588x over previous
import functools

import jax
import jax.numpy as jnp
from jax import lax
from jax.experimental import pallas as pl
from jax.experimental.pallas import tpu as pltpu
from jax.experimental.pallas import tpu_sc as plsc

NUM_TOKENS = 1000000
D = 64
B = 4096
L = 200

NC = 2
NS = 16
NW = NC * NS
B_PER_W = B // NW
CB = 4
N_CHUNKS = B_PER_W // CB
IDX_PER_CHUNK = CB * L
LP = 256
GROUPS = (0, 128)

_mesh = plsc.VectorSubcoreMesh(core_axis_name="c", subcore_axis_name="s")


@functools.partial(
    pl.kernel,
    mesh=_mesh,
    out_type=jax.ShapeDtypeStruct((B, D), jnp.float32),
    compiler_params=pltpu.CompilerParams(use_tc_tiling_on_sc=False),
    scratch_types=[
        pltpu.VMEM((CB, LP), jnp.int32),
        pltpu.VMEM((CB, LP), jnp.int32),
        pltpu.VMEM((IDX_PER_CHUNK, D), jnp.float32),
        pltpu.VMEM((IDX_PER_CHUNK, D), jnp.float32),
        pltpu.VMEM((B_PER_W, D), jnp.float32),
        pltpu.SemaphoreType.DMA,
        pltpu.SemaphoreType.DMA,
    ],
)
def _emb_kernel(idx_hbm, table_hbm, out_hbm, idx0_v, idx1_v, rows0_v, rows1_v,
                out_v, sem0, sem1):
    wid = lax.axis_index("s") * NC + lax.axis_index("c")
    row0 = wid * B_PER_W

    def gathers(idx_v, rows_v, sem):
        for s in range(CB):
            for go in GROUPS:
                gs = min(L, 128 if go == 0 else L - go)
                yield (
                    table_hbm.at[idx_v.at[s, pl.ds(go, gs)]],
                    rows_v.at[pl.ds(s * L + go, gs)],
                    sem,
                )

    def stage(g, idx_v, rows_v, sem):
        pltpu.sync_copy(idx_hbm.at[pl.ds(row0 + g * CB, CB), :], idx_v)
        for s in range(CB):
            for k in range(LP // 16):
                v = idx_v[s, pl.ds(k * 16, 16)]
                q = (v & (-SB)) | ((v & (TBLK - 1)) << 1) | ((v >> 13) & 1)
                idx_v[s, pl.ds(k * 16, 16)] = q
        for args in gathers(idx_v, rows_v, sem):
            pltpu.async_copy(*args)

    def drain(idx_v, rows_v, sem):
        for args in gathers(idx_v, rows_v, sem):
            pltpu.make_async_copy(*args).wait()

    def reduce_chunk(g, rows_v):
        for s in range(CB):
            def red(t, accs, s=s):
                base = s * L + t * 8
                a0, a1, a2, a3 = accs
                for u in range(8):
                    r = base + u
                    a0 = a0 + rows_v[r, pl.ds(0, 16)]
                    a1 = a1 + rows_v[r, pl.ds(16, 16)]
                    a2 = a2 + rows_v[r, pl.ds(32, 16)]
                    a3 = a3 + rows_v[r, pl.ds(48, 16)]
                return (a0, a1, a2, a3)
            accs = lax.fori_loop(
                0, L // 8, red,
                tuple(jnp.zeros((16,), jnp.float32) for _ in range(D // 16)),
            )
            for j in range(D // 16):
                out_v[g * CB + s, pl.ds(j * 16, 16)] = accs[j]

    stage(0, idx0_v, rows0_v, sem0)

    def pair(h, _):
        g0 = h * 2
        stage(g0 + 1, idx1_v, rows1_v, sem1)
        drain(idx0_v, rows0_v, sem0)
        reduce_chunk(g0, rows0_v)

        @pl.when(h < N_CHUNKS // 2 - 1)
        def _prefetch():
            stage(g0 + 2, idx0_v, rows0_v, sem0)

        drain(idx1_v, rows1_v, sem1)
        reduce_chunk(g0 + 1, rows1_v)
        return _

    lax.fori_loop(0, N_CHUNKS // 2, pair, None)
    pltpu.sync_copy(out_v, out_hbm.at[pl.ds(wid * B_PER_W, B_PER_W)])


TBLK = 8192
SB = 2 * TBLK
TGRID = -(-NUM_TOKENS // SB)
DENSE_ROWS = TGRID * TBLK


def _transpose_body(ta_ref, tb_ref, out_ref):
    out_ref[...] = jnp.concatenate([ta_ref[...].T, tb_ref[...].T], axis=1)


def _row_major_table(table):
    tt = table.T
    t128 = pl.pallas_call(
        _transpose_body,
        grid=(TGRID,),
        in_specs=[
            pl.BlockSpec((D, TBLK), lambda i: (0, 2 * i)),
            pl.BlockSpec(
                (D, TBLK),
                lambda i: (0, jnp.minimum(2 * i + 1, NUM_TOKENS // TBLK - 1)),
            ),
        ],
        out_specs=pl.BlockSpec((TBLK, 2 * D), lambda i: (i, 0)),
        out_shape=jax.ShapeDtypeStruct((DENSE_ROWS, 2 * D), jnp.float32),
    )(tt, tt)
    return t128.reshape(2 * DENSE_ROWS, D)


def kernel(inputs, table):
    idx_pad = jnp.pad(inputs.astype(jnp.int32), ((0, 0), (0, LP - L)))
    return _emb_kernel(idx_pad, _row_major_table(table))
